# baseline (speedup 1.0000x reference)
import jax
import jax.numpy as jnp
from jax.experimental import pallas as pl

_TILE = 512


def _nearest_centroid_kernel(x_ref, cb_ref, y_ref):
    xt = x_ref[...]
    cb = cb_ref[...]
    dots = jax.lax.dot_general(
        xt, cb, (((1,), (1,)), ((), ())),
        preferred_element_type=jnp.float32)
    c_sq = jnp.sum(cb * cb, axis=1)
    x_sq = jnp.sum(xt * xt, axis=1, keepdims=True)
    dists = x_sq - 2.0 * dots + c_sq[None, :]
    idx = jnp.argmin(dists, axis=1).astype(jnp.int32)
    y_ref[...] = idx.reshape(1, 1, _TILE)


def kernel(x, codebook):
    B, T, D = x.shape
    K = codebook.shape[0]
    bt = B * T
    num_tiles = bt // _TILE
    flat_x = x.reshape(bt, D)
    y_tiles = pl.pallas_call(
        _nearest_centroid_kernel,
        grid=(num_tiles,),
        in_specs=[
            pl.BlockSpec((_TILE, D), lambda i: (i, 0)),
            pl.BlockSpec((K, D), lambda i: (0, 0)),
        ],
        out_specs=pl.BlockSpec((1, 1, _TILE), lambda i: (i, 0, 0)),
        out_shape=jax.ShapeDtypeStruct((num_tiles, 1, _TILE), jnp.int32),
    )(flat_x, codebook)
    y = y_tiles.reshape(B, T)
    return (x, y)


# speedup vs baseline: 1.0427x; 1.0427x over previous
import jax
import jax.numpy as jnp
from jax.experimental import pallas as pl

_TILE = 512


def _nearest_centroid_kernel(xt_ref, cb_ref, xsq_ref, csq_ref, y_ref):
    xtt = xt_ref[...]
    cb = cb_ref[...]
    dots = jax.lax.dot_general(
        cb, xtt, (((1,), (0,)), ((), ())),
        preferred_element_type=jnp.float32)
    x_sq = xsq_ref[0]
    c_sq = csq_ref[...]
    dists = x_sq - 2.0 * dots + c_sq
    mins = jnp.min(dists, axis=0)
    row_iota = jax.lax.broadcasted_iota(jnp.int32, dists.shape, 0)
    K = dists.shape[0]
    cand = jnp.where(dists == mins[None, :], row_iota, K)
    idx = jnp.min(cand, axis=0).astype(jnp.int32)
    y_ref[...] = idx.reshape(1, 1, _TILE)


def kernel(x, codebook):
    B, T, D = x.shape
    K = codebook.shape[0]
    bt = B * T
    num_tiles = bt // _TILE
    flat_x = x.reshape(bt, D)
    x_t = flat_x.T
    x_sq = jnp.sum(flat_x * flat_x, axis=1, keepdims=True)
    c_sq = jnp.sum(codebook * codebook, axis=1)
    x_sq_t = x_sq.reshape(num_tiles, 1, _TILE)
    c_sq_c = c_sq.reshape(K, 1)
    y_tiles = pl.pallas_call(
        _nearest_centroid_kernel,
        grid=(num_tiles,),
        in_specs=[
            pl.BlockSpec((D, _TILE), lambda i: (0, i)),
            pl.BlockSpec((K, D), lambda i: (0, 0)),
            pl.BlockSpec((1, 1, _TILE), lambda i: (i, 0, 0)),
            pl.BlockSpec((K, 1), lambda i: (0, 0)),
        ],
        out_specs=pl.BlockSpec((1, 1, _TILE), lambda i: (i, 0, 0)),
        out_shape=jax.ShapeDtypeStruct((num_tiles, 1, _TILE), jnp.int32),
    )(x_t, codebook, x_sq_t, c_sq_c)
    y = y_tiles.reshape(B, T)
    return (x, y)
